# baseline (speedup 1.0000x reference)
import functools

import jax
import jax.numpy as jnp
from jax import lax
from jax.experimental import pallas as pl
from jax.experimental.pallas import tpu as pltpu
from jax.experimental.pallas import tpu_sc as plsc

N = 10000
E = 160000
DC = 128
NACC = 10112
SUBROWS = NACC // 16
KB = 80
EPS = E // 16
NB = EPS // KB

_MESH = plsc.VectorSubcoreMesh(
    core_axis_name="c", subcore_axis_name="s", num_cores=2, num_subcores=16
)


def _make_prop(C):
    cpc = C // 2
    shared = C % 2 == 1
    nout = C + 1 if shared else C

    @functools.partial(
        pl.kernel,
        out_type=jax.ShapeDtypeStruct((nout, NACC, DC), jnp.float32),
        mesh=_MESH,
        scratch_types=[
            pltpu.VMEM((1, EPS), jnp.int32),
            pltpu.VMEM((1, EPS), jnp.int32),
            pltpu.VMEM((KB,), jnp.int32),
            pltpu.VMEM((KB,), jnp.int32),
            pltpu.VMEM((KB,), jnp.int32),
            pltpu.VMEM((KB,), jnp.int32),
            pltpu.VMEM((KB, DC), jnp.float32),
            pltpu.VMEM((KB, DC), jnp.float32),
            pltpu.VMEM_SHARED((NACC, DC), jnp.float32),
            pltpu.SemaphoreType.DMA,
            pltpu.SemaphoreType.DMA,
            pltpu.SemaphoreType.DMA,
            pltpu.SemaphoreType.DMA,
        ],
    )
    def prop(
        zflat, srcr, dstr, zeros, out,
        src_v, dst_v, sidx0, sidx1, didx0, didx1, rows0, rows1, acc,
        gsem0, gsem1, ssem0, ssem1,
    ):
        c = lax.axis_index("c")
        s = lax.axis_index("s")
        dummy = zeros.at[pl.ds(0, KB)]
        pltpu.sync_copy(srcr.at[s], src_v)
        pltpu.sync_copy(dstr.at[s], dst_v)

        def build(b, off, sidx, didx):
            for j in range(KB // 16):
                sl = pl.ds(b * KB + j * 16, 16)
                d = pl.ds(j * 16, 16)
                sidx[d] = src_v[0, sl] + off
                didx[d] = dst_v[0, sl]

        def run_chunk(chunk, oslot, blo, npair, last):
            off = chunk * N
            pltpu.sync_copy(zeros, acc.at[pl.ds(s * SUBROWS, SUBROWS)])
            plsc.subcore_barrier()

            def body(u, _):
                @pl.when(u > 0)
                def _():
                    pltpu.make_async_copy(dummy, rows0, ssem0).wait()
                    pltpu.make_async_copy(dummy, rows1, ssem1).wait()

                build(blo + 2 * u, off, sidx0, didx0)
                g0 = pltpu.async_copy(zflat.at[sidx0], rows0, gsem0)
                build(blo + 2 * u + 1, off, sidx1, didx1)
                g1 = pltpu.async_copy(zflat.at[sidx1], rows1, gsem1)
                g0.wait()
                pltpu.async_copy(rows0, acc.at[didx0], ssem0, add=True)
                g1.wait()
                pltpu.async_copy(rows1, acc.at[didx1], ssem1, add=True)
                return 0

            lax.fori_loop(0, npair, body, 0)
            pltpu.make_async_copy(dummy, rows0, ssem0).wait()
            pltpu.make_async_copy(dummy, rows1, ssem1).wait()
            if last is not None:

                @pl.when(last >= 0)
                def _():
                    build(last, off, sidx0, didx0)
                    pltpu.async_copy(zflat.at[sidx0], rows0, gsem0).wait()
                    pltpu.sync_copy(rows0, acc.at[didx0], add=True)

            plsc.subcore_barrier()
            pltpu.sync_copy(
                acc.at[pl.ds(s * SUBROWS, SUBROWS)],
                out.at[oslot, pl.ds(s * SUBROWS, SUBROWS)],
            )

        for ci in range(cpc):
            chunk = c * cpc + ci
            run_chunk(chunk, chunk, 0, NB // 2, NB - 1)
        if shared:
            half = NB // 2 // 2 * 2
            npair = jnp.where(c == 0, half // 2, (NB - 1 - half) // 2)
            last = jnp.where(c == 0, -1, NB - 1)
            run_chunk(C - 1, C - 1 + c, c * half, npair, last)

    return prop


_PROP = {2: _make_prop(2), 5: _make_prop(5), 9: _make_prop(9)}


def _matmul(x, w, c2):
    C = x.shape[0]
    bm = 1000
    cb = 3 if c2 % 3 == 0 else c2
    bn = cb * DC
    kb = 3 if C % 3 == 0 else C
    nk = C // kb
    grid = (N // bm, c2 // cb, nk)

    def mm(x_ref, w_ref, o_ref, acc_ref):
        kk = pl.program_id(2)

        @pl.when(kk == 0)
        def _():
            acc_ref[...] = jnp.zeros_like(acc_ref)

        for ch in range(kb):
            acc_ref[...] += jnp.dot(
                x_ref[ch], w_ref[ch], preferred_element_type=jnp.float32
            )

        @pl.when(kk == nk - 1)
        def _():
            for ch in range(cb):
                o_ref[ch] = acc_ref[:, ch * DC:(ch + 1) * DC]

    return pl.pallas_call(
        mm,
        grid=grid,
        in_specs=[
            pl.BlockSpec((kb, bm, DC), lambda i, j, kk: (kk, i, 0)),
            pl.BlockSpec((kb, DC, bn), lambda i, j, kk: (kk, 0, j)),
        ],
        out_specs=pl.BlockSpec((cb, bm, DC), lambda i, j, kk: (j, i, 0)),
        out_shape=jax.ShapeDtypeStruct((c2, N, DC), jnp.float32),
        scratch_shapes=[pltpu.VMEM((bm, bn), jnp.float32)],
        compiler_params=pltpu.CompilerParams(
            dimension_semantics=("parallel", "parallel", "arbitrary")
        ),
    )(x.astype(jnp.bfloat16), w.astype(jnp.bfloat16))


def _cheb_layer(h, dis, srcr, dstr, zeros, Ws, bias, c2):
    K = Ws.shape[0]
    C = h.shape[0]
    prop = _PROP[C]
    disb = dis[None, :, None]

    def do_prop(t):
        mc = prop((disb * t).reshape(C * N, DC), srcr, dstr, zeros)
        m = mc[:, :N, :]
        if C % 2 == 1:
            m = jnp.concatenate([m[:C - 1], (m[C - 1] + m[C])[None]], axis=0)
        return m

    def wk(k):
        return Ws[k].reshape(C, DC, c2 * DC)

    out = bias + _matmul(h, wk(0), c2)
    tx1 = -disb * do_prop(h)
    out = out + _matmul(tx1, wk(1), c2)
    tx_prev, tx_pp = tx1, h
    for k in range(2, K):
        tx = -2.0 * disb * do_prop(tx_prev) - tx_pp
        out = out + _matmul(tx, wk(k), c2)
        tx_pp, tx_prev = tx_prev, tx
    return jnp.maximum(out, 0.0)


def _pad_w(Ws, din_pad, dout_pad):
    K, din, dout = Ws.shape
    return jnp.pad(Ws, ((0, 0), (0, din_pad - din), (0, dout_pad - dout)))


def _pad_b(b, dout_pad):
    return jnp.pad(b, (0, dout_pad - b.shape[0])).reshape(-1, 1, DC)


def kernel(x, edge_index, W1, b1, W2, b2, W3, b3):
    src = edge_index[0]
    dst = edge_index[1]
    mask = src != dst
    trash = jnp.int32(N)
    src2 = jnp.where(mask, src, trash)
    dst2 = jnp.where(mask, dst, trash)
    zeros = jnp.zeros((SUBROWS, DC), jnp.float32)

    srcr = src.reshape(16, 1, EPS)
    dstr = dst2.reshape(16, 1, EPS)

    degc = _PROP[2](
        jnp.ones((N * 2, DC), jnp.float32),
        dst.reshape(16, 1, EPS),
        src2.reshape(16, 1, EPS),
        zeros,
    )
    deg = degc[0, :N, 0]
    dis = jnp.where(deg > 0, lax.rsqrt(jnp.maximum(deg, 1.0)), 0.0)

    xcm = x.reshape(N, 9, DC).transpose(1, 0, 2)
    h = _cheb_layer(xcm, dis, srcr, dstr, zeros, W1, _pad_b(b1, 1152), 9)
    h = _cheb_layer(h, dis, srcr, dstr, zeros,
                    _pad_w(W2, 1152, 640), _pad_b(b2, 640), 5)
    h = _cheb_layer(h, dis, srcr, dstr, zeros,
                    _pad_w(W3, 640, 384), _pad_b(b3, 384), 3)
    return h.transpose(1, 0, 2).reshape(N, 384)[:, :288]


# speedup vs baseline: 1.0057x; 1.0057x over previous
import functools

import jax
import jax.numpy as jnp
from jax import lax
from jax.experimental import pallas as pl
from jax.experimental.pallas import tpu as pltpu
from jax.experimental.pallas import tpu_sc as plsc

N = 10000
E = 160000
DC = 128
NACC = 10112
SUBROWS = NACC // 16
KB = 80
EPS = E // 16
NB = EPS // KB

_MESH = plsc.VectorSubcoreMesh(
    core_axis_name="c", subcore_axis_name="s", num_cores=2, num_subcores=16
)


def _make_prop(C):
    cpc = (C + 1) // 2

    @functools.partial(
        pl.kernel,
        out_type=jax.ShapeDtypeStruct((C, NACC, DC), jnp.float32),
        mesh=_MESH,
        scratch_types=[
            pltpu.VMEM((1, EPS), jnp.int32),
            pltpu.VMEM((1, EPS), jnp.int32),
            pltpu.VMEM((KB,), jnp.int32),
            pltpu.VMEM((KB,), jnp.int32),
            pltpu.VMEM((KB,), jnp.int32),
            pltpu.VMEM((KB,), jnp.int32),
            pltpu.VMEM((KB, DC), jnp.float32),
            pltpu.VMEM((KB, DC), jnp.float32),
            pltpu.VMEM_SHARED((NACC, DC), jnp.float32),
            pltpu.SemaphoreType.DMA,
            pltpu.SemaphoreType.DMA,
            pltpu.SemaphoreType.DMA,
            pltpu.SemaphoreType.DMA,
        ],
    )
    def prop(
        zflat, srcr, dstr, zeros, out,
        src_v, dst_v, sidx0, sidx1, didx0, didx1, rows0, rows1, acc,
        gsem0, gsem1, ssem0, ssem1,
    ):
        c = lax.axis_index("c")
        s = lax.axis_index("s")
        dummy = zeros.at[pl.ds(0, KB)]
        pltpu.sync_copy(srcr.at[s], src_v)
        pltpu.sync_copy(dstr.at[s], dst_v)

        def build(b, off, sidx, didx):
            for j in range(KB // 16):
                sl = pl.ds(b * KB + j * 16, 16)
                d = pl.ds(j * 16, 16)
                sidx[d] = src_v[0, sl] + off
                didx[d] = dst_v[0, sl]

        for ci in range(cpc):
            chunk = c * cpc + ci

            def chunk_body(chunk=chunk):
                off = chunk * N
                pltpu.sync_copy(zeros, acc.at[pl.ds(s * SUBROWS, SUBROWS)])
                plsc.subcore_barrier()

                def body(u, _, off=off):
                    @pl.when(u > 0)
                    def _():
                        pltpu.make_async_copy(dummy, rows0, ssem0).wait()
                        pltpu.make_async_copy(dummy, rows1, ssem1).wait()

                    build(2 * u, off, sidx0, didx0)
                    g0 = pltpu.async_copy(zflat.at[sidx0], rows0, gsem0)
                    build(2 * u + 1, off, sidx1, didx1)
                    g1 = pltpu.async_copy(zflat.at[sidx1], rows1, gsem1)
                    g0.wait()
                    pltpu.async_copy(rows0, acc.at[didx0], ssem0, add=True)
                    g1.wait()
                    pltpu.async_copy(rows1, acc.at[didx1], ssem1, add=True)
                    return 0

                lax.fori_loop(0, NB // 2, body, 0)
                pltpu.make_async_copy(dummy, rows0, ssem0).wait()
                pltpu.make_async_copy(dummy, rows1, ssem1).wait()
                build(NB - 1, off, sidx0, didx0)
                pltpu.async_copy(zflat.at[sidx0], rows0, gsem0).wait()
                pltpu.sync_copy(rows0, acc.at[didx0], add=True)
                plsc.subcore_barrier()
                pltpu.sync_copy(
                    acc.at[pl.ds(s * SUBROWS, SUBROWS)],
                    out.at[chunk, pl.ds(s * SUBROWS, SUBROWS)],
                )

            if ci < C - cpc:
                chunk_body()
            else:
                pl.when(c == 0)(chunk_body)

    return prop


_PROP = {2: _make_prop(2), 5: _make_prop(5), 9: _make_prop(9)}


def _matmul(x, w, c2):
    C = x.shape[0]
    bm = 1000
    cb = 3 if c2 % 3 == 0 else c2
    bn = cb * DC
    kb = 3 if C % 3 == 0 else C
    nk = C // kb
    grid = (N // bm, c2 // cb, nk)

    def mm(x_ref, w_ref, o_ref, acc_ref):
        kk = pl.program_id(2)

        @pl.when(kk == 0)
        def _():
            acc_ref[...] = jnp.zeros_like(acc_ref)

        for ch in range(kb):
            acc_ref[...] += jnp.dot(
                x_ref[ch], w_ref[ch], preferred_element_type=jnp.float32
            )

        @pl.when(kk == nk - 1)
        def _():
            for ch in range(cb):
                o_ref[ch] = acc_ref[:, ch * DC:(ch + 1) * DC]

    return pl.pallas_call(
        mm,
        grid=grid,
        in_specs=[
            pl.BlockSpec((kb, bm, DC), lambda i, j, kk: (kk, i, 0)),
            pl.BlockSpec((kb, DC, bn), lambda i, j, kk: (kk, 0, j)),
        ],
        out_specs=pl.BlockSpec((cb, bm, DC), lambda i, j, kk: (j, i, 0)),
        out_shape=jax.ShapeDtypeStruct((c2, N, DC), jnp.float32),
        scratch_shapes=[pltpu.VMEM((bm, bn), jnp.float32)],
        compiler_params=pltpu.CompilerParams(
            dimension_semantics=("parallel", "parallel", "arbitrary")
        ),
    )(x.astype(jnp.bfloat16), w.astype(jnp.bfloat16))


def _cheb_layer(h, dis, srcr, dstr, zeros, Ws, bias, c2):
    K = Ws.shape[0]
    C = h.shape[0]
    prop = _PROP[C]
    disb = dis[None, :, None]

    def do_prop(t):
        mc = prop((disb * t).reshape(C * N, DC), srcr, dstr, zeros)
        return mc[:, :N, :]

    def wk(k):
        return Ws[k].reshape(C, DC, c2 * DC)

    out = bias + _matmul(h, wk(0), c2)
    tx1 = -disb * do_prop(h)
    out = out + _matmul(tx1, wk(1), c2)
    tx_prev, tx_pp = tx1, h
    for k in range(2, K):
        tx = -2.0 * disb * do_prop(tx_prev) - tx_pp
        out = out + _matmul(tx, wk(k), c2)
        tx_pp, tx_prev = tx_prev, tx
    return jnp.maximum(out, 0.0)


def _pad_w(Ws, din_pad, dout_pad):
    K, din, dout = Ws.shape
    return jnp.pad(Ws, ((0, 0), (0, din_pad - din), (0, dout_pad - dout)))


def _pad_b(b, dout_pad):
    return jnp.pad(b, (0, dout_pad - b.shape[0])).reshape(-1, 1, DC)


def kernel(x, edge_index, W1, b1, W2, b2, W3, b3):
    src = edge_index[0]
    dst = edge_index[1]
    mask = src != dst
    trash = jnp.int32(N)
    src2 = jnp.where(mask, src, trash)
    dst2 = jnp.where(mask, dst, trash)
    zeros = jnp.zeros((SUBROWS, DC), jnp.float32)

    srcr = src.reshape(16, 1, EPS)
    dstr = dst2.reshape(16, 1, EPS)

    degc = _PROP[2](
        jnp.ones((N * 2, DC), jnp.float32),
        dst.reshape(16, 1, EPS),
        src2.reshape(16, 1, EPS),
        zeros,
    )
    deg = degc[0, :N, 0]
    dis = jnp.where(deg > 0, lax.rsqrt(jnp.maximum(deg, 1.0)), 0.0)

    xcm = x.reshape(N, 9, DC).transpose(1, 0, 2)
    h = _cheb_layer(xcm, dis, srcr, dstr, zeros, W1, _pad_b(b1, 1152), 9)
    h = _cheb_layer(h, dis, srcr, dstr, zeros,
                    _pad_w(W2, 1152, 640), _pad_b(b2, 640), 5)
    h = _cheb_layer(h, dis, srcr, dstr, zeros,
                    _pad_w(W3, 640, 384), _pad_b(b3, 384), 3)
    return h.transpose(1, 0, 2).reshape(N, 384)[:, :288]
